# baseline (speedup 1.0000x reference)
import functools

import jax
import jax.numpy as jnp
from jax.experimental import pallas as pl
from jax.experimental.pallas import tpu as pltpu

_CHUNK = 128
_CUT = 0.05
_BIG = 1e30


def _norm_kernel(mcol_ref, ei_ref, ej_ref, eo_ref, fo_ref):
    m = mcol_ref[0].astype(jnp.float32)
    for src, dst in ((ei_ref, eo_ref), (ej_ref, fo_ref)):
        e = src[...]
        dst[...] = e * (m / jnp.maximum(
            jnp.sqrt(jnp.sum(e * e, axis=1, keepdims=True)), 1e-8))


def _lr_kernel(ei_ref, ej_ref, xj_ref, xi_ref, out_ref,
               wbuf_ref, accy_ref, num_ref, thr_ref, *, blk, batch, nb):
    i = pl.program_id(0)
    j = pl.program_id(1)
    p = jax.lax.rem(i * (nb + 1) + j, 2)

    @pl.when((i == 0) & (j == 0))
    def _boot():
        wbuf_ref[1] = jnp.zeros_like(wbuf_ref[1])
        r = jax.lax.broadcasted_iota(jnp.int32, (blk, blk), 0)
        c = jax.lax.broadcasted_iota(jnp.int32, (blk, blk), 1)
        d = r - c
        for k, t in ((0, -1), (1, 0), (2, 1)):
            thr_ref[k] = jnp.where(jnp.abs(t * blk + d) > _CHUNK, _CUT, _BIG)
        thr_ref[3] = jnp.full((blk, blk), _CUT, jnp.float32)
        thr_ref[4] = jnp.full((blk, blk), _BIG, jnp.float32)

    @pl.when(j == 0)
    def _zero_row():
        num_ref[...] = jnp.zeros_like(num_ref)
        accy_ref[...] = jnp.zeros_like(accy_ref)

    w_prev = wbuf_ref[1 - p]
    for b in range(batch):
        accy_ref[b] += jnp.dot(w_prev, xj_ref[b],
                               preferred_element_type=jnp.float32)

    s = jnp.abs(jax.lax.dot_general(
        ei_ref[...], ej_ref[...], (((1,), (1,)), ((), ())),
        preferred_element_type=jnp.float32))
    t = i - j
    k = jnp.where(j >= nb, 4, jnp.where(jnp.abs(t) <= 1, t + 1, 3))
    keepf = jnp.where(s > thr_ref[k], 1.0, 0.0)
    wbuf_ref[p] = s * keepf
    num_ref[...] += jnp.sum(keepf, axis=1, keepdims=True)

    @pl.when(j == nb)
    def _fin():
        num = num_ref[...]
        xi = xi_ref[...]
        y = accy_ref[...] / jnp.maximum(num, 1.0)[None]
        out_ref[...] = jnp.where((num > 0.0)[None], (xi + y) * 0.5, xi)


@jax.jit
def kernel(x, mask, emb_i_weight, emb_j_weight):
    B, L, D = x.shape
    E = emb_i_weight.shape[1]
    blk = 512 if L % 512 == 0 else 128
    nb = L // blk
    mask_col = mask.reshape(nb, blk, 1)
    einorm, ejnorm = pl.pallas_call(
        _norm_kernel,
        grid=(nb,),
        in_specs=[
            pl.BlockSpec((1, blk, 1), lambda n: (n, 0, 0)),
            pl.BlockSpec((blk, E), lambda n: (n, 0)),
            pl.BlockSpec((blk, E), lambda n: (n, 0)),
        ],
        out_specs=[
            pl.BlockSpec((blk, E), lambda n: (n, 0)),
            pl.BlockSpec((blk, E), lambda n: (n, 0)),
        ],
        out_shape=[
            jax.ShapeDtypeStruct((L, E), jnp.float32),
            jax.ShapeDtypeStruct((L, E), jnp.float32),
        ],
    )(mask_col, emb_i_weight, emb_j_weight)
    return pl.pallas_call(
        functools.partial(_lr_kernel, blk=blk, batch=B, nb=nb),
        grid=(nb, nb + 1),
        in_specs=[
            pl.BlockSpec((blk, E), lambda i, j: (i, 0)),
            pl.BlockSpec((blk, E), lambda i, j: (jnp.minimum(j, nb - 1), 0)),
            pl.BlockSpec((B, blk, D),
                         lambda i, j: (0, jnp.maximum(j, 1) - 1, 0)),
            pl.BlockSpec((B, blk, D), lambda i, j: (0, i, 0)),
        ],
        out_specs=pl.BlockSpec((B, blk, D), lambda i, j: (0, i, 0)),
        out_shape=jax.ShapeDtypeStruct((B, L, D), x.dtype),
        scratch_shapes=[
            pltpu.VMEM((2, blk, blk), jnp.float32),
            pltpu.VMEM((B, blk, D), jnp.float32),
            pltpu.VMEM((blk, 1), jnp.float32),
            pltpu.VMEM((5, blk, blk), jnp.float32),
        ],
        compiler_params=pltpu.CompilerParams(
            dimension_semantics=("arbitrary", "arbitrary")),
    )(einorm, ejnorm, x, x)
